# Initial kernel scaffold; baseline (speedup 1.0000x reference)
import functools

import jax
import jax.numpy as jnp
from jax import lax
from jax.experimental import pallas as pl
from jax.experimental.pallas import tpu as pltpu
from jax.experimental.pallas import tpu_sc as plsc

NUM_TYPES = 1000
SEQ_LEN = 3276800
NC = 2
NS = 16
NW = NC * NS
CHUNK = SEQ_LEN // NW
BLK = 10240
NBLK = CHUNK // BLK
ZSLOT = NUM_TYPES * NUM_TYPES
WPAD = ZSLOT + 16


def _body(seq_h, w_h, pad_h, out_h, w_sh, buf, idxb, valb, padv, sem):
    c = lax.axis_index("c")
    s = lax.axis_index("s")
    wid = s * NC + c
    base = wid * CHUNK

    @pl.when(s == 0)
    def _():
        pltpu.sync_copy(w_h, w_sh.at[pl.ds(0, ZSLOT)])
        valb[pl.ds(0, 16)] = jnp.zeros((16,), jnp.float32)
        pltpu.sync_copy(valb.at[pl.ds(0, 16)], w_sh.at[pl.ds(ZSLOT, 16)])

    pltpu.sync_copy(pad_h, padv)
    pltpu.sync_copy(seq_h.at[pl.ds(base, CHUNK)], buf.at[pl.ds(0, CHUNK)])

    @pl.when(wid != NW - 1)
    def _():
        pltpu.sync_copy(seq_h.at[pl.ds(base + CHUNK, 16)],
                        buf.at[pl.ds(CHUNK, 16)])

    pad = padv[...]

    @pl.when(wid == NW - 1)
    def _():
        buf[pl.ds(CHUNK, 16)] = pad

    plsc.subcore_barrier()

    def sub(j, acc):
        o = j * BLK

        def compute(i, carry):
            a = buf[pl.ds(o + i * 16, 16)]
            b = buf[pl.ds(o + i * 16 + 1, 16)]
            m = (a != pad) & (b != pad)
            fi = a * NUM_TYPES + b
            idxb[pl.ds(i * 16, 16)] = jnp.where(m, fi, ZSLOT)
            return carry

        lax.fori_loop(0, BLK // 16, compute, 0)
        pltpu.async_copy(w_sh.at[idxb], valb, sem).wait()

        def accum(i, a2):
            return a2 + valb[pl.ds(i * 16, 16)]

        return lax.fori_loop(0, BLK // 16, accum, acc)

    acc = lax.fori_loop(0, NBLK, sub, jnp.zeros((16,), jnp.float32))
    valb[pl.ds(0, 16)] = acc
    pltpu.sync_copy(valb.at[pl.ds(0, 16)], out_h.at[wid])


@functools.partial(
    pl.kernel,
    out_type=jax.ShapeDtypeStruct((NW, 16), jnp.float32),
    mesh=plsc.VectorSubcoreMesh(core_axis_name="c", subcore_axis_name="s"),
    scratch_types=[
        pltpu.VMEM_SHARED((WPAD,), jnp.float32),
        pltpu.VMEM((CHUNK + 16,), jnp.int32),
        pltpu.VMEM((BLK,), jnp.int32),
        pltpu.VMEM((BLK,), jnp.float32),
        pltpu.VMEM((16,), jnp.int32),
        pltpu.SemaphoreType.DMA,
    ],
)
def _partials(seq_h, w_h, pad_h, out_h, w_sh, buf, idxb, valb, padv, sem):
    _body(seq_h, w_h, pad_h, out_h, w_sh, buf, idxb, valb, padv, sem)


def kernel(sequence, padding_idx, W):
    padv = jnp.full((16,), padding_idx, dtype=jnp.int32)
    parts = _partials(sequence, W.reshape(-1), padv)
    return -jnp.sum(parts)


# speedup vs baseline: 447.9528x; 447.9528x over previous
import functools

import jax
import jax.numpy as jnp
from jax import lax
from jax.experimental import pallas as pl
from jax.experimental.pallas import tpu as pltpu
from jax.experimental.pallas import tpu_sc as plsc

NUM_TYPES = 1000
SEQ_LEN = 3276800
NC = 2
NS = 16
NW = NC * NS
CHUNK = SEQ_LEN // NW
BLK = 10240
NBLK = CHUNK // BLK
ZSLOT = NUM_TYPES * NUM_TYPES
WPAD = ZSLOT + 16


def _body(seq_h, w_h, pad_h, out_h, w_sh, buf, idxb, valb, padv, sem):
    c = lax.axis_index("c")
    s = lax.axis_index("s")
    wid = s * NC + c
    base = wid * CHUNK

    @pl.when(s == 0)
    def _():
        pltpu.sync_copy(w_h, w_sh)

    pltpu.sync_copy(pad_h, padv)
    pad = padv[...]
    plsc.subcore_barrier()

    def sub(j, acc):
        o = base + j * BLK
        last = (wid == NW - 1) & (j == NBLK - 1)

        @pl.when(jnp.logical_not(last))
        def _():
            pltpu.sync_copy(seq_h.at[pl.ds(o, BLK + 16)], buf)

        @pl.when(last)
        def _():
            pltpu.sync_copy(seq_h.at[pl.ds(o, BLK)], buf.at[pl.ds(0, BLK)])
            buf[pl.ds(BLK, 16)] = pad

        def compute(i, carry):
            a = buf[pl.ds(i * 16, 16)]
            b = buf[pl.ds(i * 16 + 1, 16)]
            m = (a != pad) & (b != pad)
            fi = a * NUM_TYPES + b
            idxb[pl.ds(i * 16, 16)] = jnp.where(m, fi, ZSLOT)
            return carry

        lax.fori_loop(0, BLK // 16, compute, 0)
        pltpu.async_copy(w_sh.at[idxb], valb, sem).wait()

        def accum(i, a2):
            return a2 + valb[pl.ds(i * 16, 16)]

        return lax.fori_loop(0, BLK // 16, accum, acc)

    acc = lax.fori_loop(0, NBLK, sub, jnp.zeros((16,), jnp.float32))
    valb[pl.ds(0, 16)] = acc
    pltpu.sync_copy(valb.at[pl.ds(0, 16)], out_h.at[pl.ds(wid * 16, 16)])


@functools.partial(
    pl.kernel,
    out_type=jax.ShapeDtypeStruct((NW * 16,), jnp.float32),
    mesh=plsc.VectorSubcoreMesh(core_axis_name="c", subcore_axis_name="s"),
    scratch_types=[
        pltpu.VMEM_SHARED((WPAD,), jnp.float32),
        pltpu.VMEM((BLK + 16,), jnp.int32),
        pltpu.VMEM((BLK,), jnp.int32),
        pltpu.VMEM((BLK,), jnp.float32),
        pltpu.VMEM((16,), jnp.int32),
        pltpu.SemaphoreType.DMA,
    ],
)
def _partials(seq_h, w_h, pad_h, out_h, w_sh, buf, idxb, valb, padv, sem):
    _body(seq_h, w_h, pad_h, out_h, w_sh, buf, idxb, valb, padv, sem)


def kernel(sequence, padding_idx, W):
    padv = jnp.full((16,), padding_idx, dtype=jnp.int32)
    wpad = jnp.concatenate(
        [W.reshape(-1), jnp.zeros((WPAD - ZSLOT,), jnp.float32)])
    parts = _partials(sequence, wpad, padv)
    return -jnp.sum(parts)


# speedup vs baseline: 735.3451x; 1.6416x over previous
import functools

import jax
import jax.numpy as jnp
from jax import lax
from jax.experimental import pallas as pl
from jax.experimental.pallas import tpu as pltpu
from jax.experimental.pallas import tpu_sc as plsc

NUM_TYPES = 1000
SEQ_LEN = 3276800
NC = 2
NS = 16
NW = NC * NS
CHUNK = SEQ_LEN // NW
BLK = 10240
NBLK = CHUNK // BLK
ZSLOT = NUM_TYPES * NUM_TYPES
WPAD = ZSLOT + 16


def _body(seq_h, w_h, pad_h, out_h,
          w_sh, buf0, buf1, idx0, idx1, val0, val1, padv,
          seq_sem, gat_sem, w_sem):
    c = lax.axis_index("c")
    s = lax.axis_index("s")
    wid = s * NC + c
    base = wid * CHUNK
    bufs, idxs, vals = (buf0, buf1), (idx0, idx1), (val0, val1)
    islast = wid == NW - 1

    @pl.when(s == 0)
    def _():
        pltpu.make_async_copy(w_h, w_sh, w_sem).start()

    pltpu.sync_copy(pad_h, padv)
    pad = padv[...]

    def issue_seq(j):
        b = bufs[j % 2]
        o = base + j * BLK
        if j < NBLK - 1:
            pltpu.make_async_copy(seq_h.at[pl.ds(o, BLK + 16)], b,
                                  seq_sem).start()
        else:
            @pl.when(islast)
            def _():
                pltpu.make_async_copy(seq_h.at[pl.ds(o, BLK)],
                                      b.at[pl.ds(0, BLK)], seq_sem).start()

            @pl.when(jnp.logical_not(islast))
            def _():
                pltpu.make_async_copy(seq_h.at[pl.ds(o, BLK + 16)], b,
                                      seq_sem).start()

    def wait_seq(j):
        b = bufs[j % 2]
        o = base + j * BLK
        if j < NBLK - 1:
            pltpu.make_async_copy(seq_h.at[pl.ds(o, BLK + 16)], b,
                                  seq_sem).wait()
        else:
            @pl.when(islast)
            def _():
                pltpu.make_async_copy(seq_h.at[pl.ds(o, BLK)],
                                      b.at[pl.ds(0, BLK)], seq_sem).wait()
                b[pl.ds(BLK, 16)] = pad

            @pl.when(jnp.logical_not(islast))
            def _():
                pltpu.make_async_copy(seq_h.at[pl.ds(o, BLK + 16)], b,
                                      seq_sem).wait()

    def gather(j):
        return pltpu.make_async_copy(w_sh.at[idxs[j % 2]], vals[j % 2],
                                     gat_sem)

    def merged(j, acc, accumulate):
        b, ij = bufs[j % 2], idxs[j % 2]
        vprev = vals[j % 2]

        @plsc.parallel_loop(0, BLK, step=16, unroll=4, carry=acc)
        def acc(i, a3):
            a = b[pl.ds(i, 16)]
            nxt = b[pl.ds(i + 1, 16)]
            m = (a != pad) & (nxt != pad)
            fi = a * NUM_TYPES + nxt
            ij[pl.ds(i, 16)] = jnp.where(m, fi, ZSLOT)
            if accumulate:
                a3 = a3 + vprev[pl.ds(i, 16)]
            return a3

        return acc

    def accum_tail(j, acc):
        v = vals[j % 2]

        @plsc.parallel_loop(0, BLK, step=16, unroll=4, carry=acc)
        def acc(i, a3):
            return a3 + v[pl.ds(i, 16)]

        return acc

    issue_seq(0)
    acc = jnp.zeros((16,), jnp.float32)
    for j in range(NBLK):
        wait_seq(j)
        if j + 1 < NBLK:
            issue_seq(j + 1)
        acc = merged(j, acc, accumulate=(j >= 2))
        if j == 0:
            @pl.when(s == 0)
            def _():
                pltpu.make_async_copy(w_h, w_sh, w_sem).wait()

            plsc.subcore_barrier()
        if j >= 1:
            gather(j - 1).wait()
        gather(j).start()
    gather(NBLK - 1).wait()
    acc = accum_tail(NBLK - 2, acc)
    acc = accum_tail(NBLK - 1, acc)

    val0[pl.ds(0, 16)] = acc
    pltpu.sync_copy(val0.at[pl.ds(0, 16)], out_h.at[pl.ds(wid * 16, 16)])


@functools.partial(
    pl.kernel,
    out_type=jax.ShapeDtypeStruct((NW * 16,), jnp.float32),
    mesh=plsc.VectorSubcoreMesh(core_axis_name="c", subcore_axis_name="s"),
    scratch_types=[
        pltpu.VMEM_SHARED((WPAD,), jnp.float32),
        pltpu.VMEM((BLK + 16,), jnp.int32),
        pltpu.VMEM((BLK + 16,), jnp.int32),
        pltpu.VMEM((BLK,), jnp.int32),
        pltpu.VMEM((BLK,), jnp.int32),
        pltpu.VMEM((BLK,), jnp.float32),
        pltpu.VMEM((BLK,), jnp.float32),
        pltpu.VMEM((16,), jnp.int32),
        pltpu.SemaphoreType.DMA,
        pltpu.SemaphoreType.DMA,
        pltpu.SemaphoreType.DMA,
    ],
)
def _partials(seq_h, w_h, pad_h, out_h, *rest):
    _body(seq_h, w_h, pad_h, out_h, *rest)


def kernel(sequence, padding_idx, W):
    padv = jnp.full((16,), padding_idx, dtype=jnp.int32)
    wpad = jnp.concatenate(
        [W.reshape(-1), jnp.zeros((WPAD - ZSLOT,), jnp.float32)])
    parts = _partials(sequence, wpad, padv)
    return -jnp.sum(parts)
